# Initial kernel scaffold; baseline (speedup 1.0000x reference)
import functools

import jax
import jax.numpy as jnp
from jax.experimental import pallas as pl
from jax.experimental.pallas import tpu as pltpu
from jax.experimental.pallas import tpu_sc as plsc

_LANES = 16


@jax.jit
def kernel(x, token_table, pos_table):
    B, L = x.shape
    V, D = token_table.shape
    N = B * L
    W = 8 * L

    x_flat = x.reshape(1, N).astype(jnp.int32)

    @functools.partial(
        pl.kernel,
        out_type=jax.ShapeDtypeStruct((N, D), jnp.float32),
        mesh=plsc.VectorSubcoreMesh(
            core_axis_name="core", subcore_axis_name="subcore"
        ),
        scratch_types=[pltpu.VMEM((L, D), jnp.float32)],
    )
    def sc_embed(tok_hbm, idx_hbm, pos_hbm, out_hbm, pos_vmem):
        pltpu.sync_copy(pos_hbm, pos_vmem)

        def body(i_vmem, o_vmem):
            pltpu.sync_copy(tok_hbm.at[i_vmem.at[0]], o_vmem)

            @pl.loop(0, L)
            def _(l):
                for c in range(0, D, _LANES):
                    p = pos_vmem[l, pl.ds(c, _LANES)]
                    for s in range(W // L):
                        plsc.addupdate(
                            o_vmem.at[s * L + l, pl.ds(c, _LANES)], p
                        )

        pltpu.emit_pipeline(
            body,
            grid=(N // W,),
            in_specs=[pl.BlockSpec((1, W), lambda i: (0, i))],
            out_specs=[pl.BlockSpec((W, D), lambda i: (i, 0))],
            core_axis_name=("core", "subcore"),
            dimension_semantics=(pltpu.PARALLEL,),
        )(idx_hbm, out_hbm)

    out = sc_embed(token_table, x_flat, pos_table)
    return out.reshape(B, L, D)


# speedup vs baseline: 1.4389x; 1.4389x over previous
import functools

import jax
import jax.numpy as jnp
from jax.experimental import pallas as pl
from jax.experimental.pallas import tpu as pltpu
from jax.experimental.pallas import tpu_sc as plsc

_LANES = 16


@jax.jit
def kernel(x, token_table, pos_table):
    B, L = x.shape
    V, D = token_table.shape
    N = B * L
    W = 8 * L

    x_flat = x.reshape(N).astype(jnp.int32)

    @functools.partial(
        pl.kernel,
        out_type=jax.ShapeDtypeStruct((N, D), jnp.float32),
        mesh=plsc.VectorSubcoreMesh(
            core_axis_name="core", subcore_axis_name="subcore"
        ),
        scratch_types=[pltpu.VMEM((L, D), jnp.float32)],
        compiler_params=pltpu.CompilerParams(use_tc_tiling_on_sc=False),
    )
    def sc_embed(tok_hbm, idx_hbm, pos_hbm, out_hbm, pos_vmem):
        pltpu.sync_copy(pos_hbm, pos_vmem)

        def body(i_vmem, o_vmem):
            pltpu.sync_copy(tok_hbm.at[i_vmem], o_vmem)

            @pl.loop(0, L)
            def _(l):
                for c in range(0, D, _LANES):
                    p = pos_vmem[l, pl.ds(c, _LANES)]
                    for s in range(W // L):
                        plsc.addupdate(
                            o_vmem.at[s * L + l, pl.ds(c, _LANES)], p
                        )

        pltpu.emit_pipeline(
            body,
            grid=(N // W,),
            in_specs=[pl.BlockSpec((W,), lambda i: (i,))],
            out_specs=[pl.BlockSpec((W, D), lambda i: (i, 0))],
            core_axis_name=("core", "subcore"),
            dimension_semantics=(pltpu.PARALLEL,),
        )(idx_hbm, out_hbm)

    out = sc_embed(token_table, x_flat, pos_table)
    return out.reshape(B, L, D)
